# baseline (speedup 1.0000x reference)
import jax
import jax.numpy as jnp
from jax import lax
from jax.experimental import pallas as pl
from jax.experimental.pallas import tpu as pltpu
from jax.experimental.pallas import tpu_sc as plsc

BATCH = 16384
EMB = 64
LANES = 16
NC = 2
NS = 16
NW = NC * NS
BPW = BATCH // NW
CHUNK = 128
NCHUNK = BPW // CHUNK


def _sc_body(u_hbm, pos_hbm, neg_hbm, uw_hbm, iw_hbm, out_hbm,
             u_idx, p_idx, n_idx, u_rows, p_rows, n_rows, part, sem):
    wid = lax.axis_index("s") * NC + lax.axis_index("c")
    base = wid * BPW

    pltpu.sync_copy(u_hbm.at[pl.ds(base, BPW)], u_idx)
    pltpu.sync_copy(pos_hbm.at[pl.ds(base, BPW)], p_idx)
    pltpu.sync_copy(neg_hbm.at[pl.ds(base, BPW)], n_idx)

    copies = []
    for j in range(NCHUNK):
        sl = pl.ds(j * CHUNK, CHUNK)
        copies.append(pltpu.async_copy(uw_hbm.at[u_idx.at[sl]], u_rows.at[sl], sem))
        copies.append(pltpu.async_copy(iw_hbm.at[p_idx.at[sl]], p_rows.at[sl], sem))
        copies.append(pltpu.async_copy(iw_hbm.at[n_idx.at[sl]], n_rows.at[sl], sem))
    for cp in copies:
        cp.wait()

    def body(b, carry):
        sl0 = pl.ds(0, LANES)
        acc = u_rows[b, sl0] * (p_rows[b, sl0] - n_rows[b, sl0])
        for c in range(1, EMB // LANES):
            sl = pl.ds(c * LANES, LANES)
            acc = acc + u_rows[b, sl] * (p_rows[b, sl] - n_rows[b, sl])
        part[b, :] = acc
        return carry

    lax.fori_loop(0, BPW, body, 0)
    pltpu.sync_copy(part, out_hbm.at[pl.ds(base, BPW)])


_sc_call = pl.kernel(
    _sc_body,
    mesh=plsc.VectorSubcoreMesh(core_axis_name="c", subcore_axis_name="s"),
    out_type=jax.ShapeDtypeStruct((BATCH, LANES), jnp.float32),
    scratch_types=[
        pltpu.VMEM((BPW,), jnp.int32),
        pltpu.VMEM((BPW,), jnp.int32),
        pltpu.VMEM((BPW,), jnp.int32),
        pltpu.VMEM((BPW, EMB), jnp.float32),
        pltpu.VMEM((BPW, EMB), jnp.float32),
        pltpu.VMEM((BPW, EMB), jnp.float32),
        pltpu.VMEM((BPW, LANES), jnp.float32),
        pltpu.SemaphoreType.DMA,
    ],
    compiler_params=pltpu.CompilerParams(use_tc_tiling_on_sc=False),
)


def _loss_body(part_ref, out_ref):
    x = part_ref[...]
    s = jnp.sum(x, axis=1, keepdims=True)
    l = jnp.log(jax.nn.sigmoid(s) + 1e-10)
    out_ref[0, 0] = -jnp.sum(l) * (1.0 / BATCH)


_loss_call = pl.pallas_call(
    _loss_body,
    out_shape=jax.ShapeDtypeStruct((1, 1), jnp.float32),
    out_specs=pl.BlockSpec(memory_space=pltpu.SMEM),
)


def kernel(u, pos, neg, u_emb_w, i_emb_w):
    u = u.astype(jnp.int32)
    pos = pos.astype(jnp.int32)
    neg = neg.astype(jnp.int32)
    partials = _sc_call(u, pos, neg, u_emb_w, i_emb_w)
    loss = _loss_call(partials)
    return loss[0, 0]


# speedup vs baseline: 2.1980x; 2.1980x over previous
import jax
import jax.numpy as jnp
from jax import lax
from jax.experimental import pallas as pl
from jax.experimental.pallas import tpu as pltpu
from jax.experimental.pallas import tpu_sc as plsc

BATCH = 16384
NROWS = 1000000
EMB = 64
LANES = 16
NC = 2
NS = 16
BPT = BATCH // NS
SEG = 62464
TAIL0 = SEG * NS
TAILW = 512
TAIL2 = TAIL0 + TAILW
NTAIL = NROWS - TAIL2
DPC = EMB // NC
GRP = 8
NGRP = DPC // GRP


def _sc_body(u_hbm, pos_hbm, neg_hbm, uwt_hbm, iwt_hbm, aux_u_hbm, aux_i_hbm,
             out_hbm,
             row_sh, u_idx, p_idx, n_idx, uv, pv, nv, acc,
             aux_u_v, aux_i_v, gsem):
    core = lax.axis_index("c")
    tid = lax.axis_index("s")

    bsl = pl.ds(tid * BPT, BPT)
    pltpu.sync_copy(u_hbm.at[bsl], u_idx)
    pltpu.sync_copy(pos_hbm.at[bsl], p_idx)
    pltpu.sync_copy(neg_hbm.at[bsl], n_idx)

    @pl.when(tid == NS - 1)
    def _stage_aux():
        pltpu.sync_copy(aux_u_hbm, aux_u_v)
        pltpu.sync_copy(aux_i_hbm, aux_i_v)

    def zero_body(k, carry):
        acc[pl.ds(k * LANES, LANES)] = jnp.zeros((LANES,), jnp.float32)
        return carry

    lax.fori_loop(0, BPT // LANES, zero_body, 0)

    start = pl.multiple_of(tid * SEG, 128)
    ssl = pl.ds(start, SEG)
    tsl = pl.ds(TAIL0, TAILW)
    dsl = pl.ds(TAIL2, NTAIL)
    dummy = aux_u_hbm.at[pl.ds(0, BPT)]

    def stage_row(table_hbm, aux_v, g, d8):
        pltpu.sync_copy(table_hbm.at[core, g, d8, ssl], row_sh.at[ssl])

        @pl.when(tid == NS - 1)
        def _tail():
            pltpu.sync_copy(table_hbm.at[core, g, d8, tsl], row_sh.at[tsl])
            off = pl.multiple_of((core * DPC + g * GRP + d8) * NTAIL, NTAIL)
            pltpu.sync_copy(aux_v.at[pl.ds(off, NTAIL)], row_sh.at[dsl])

        plsc.subcore_barrier()

    def grp_body(g, carry):
        for d8 in range(GRP):
            stage_row(uwt_hbm, aux_u_v, g, d8)

            def fire_u(k, c2):
                sl = pl.ds(k * 128, 128)
                pltpu.async_copy(row_sh.at[u_idx.at[sl]], uv.at[sl], gsem)
                return c2

            lax.fori_loop(0, BPT // 128, fire_u, 0)
            pltpu.make_async_copy(dummy, uv, gsem).wait()
            plsc.subcore_barrier()

            stage_row(iwt_hbm, aux_i_v, g, d8)

            def fire_i(k, c2):
                sl = pl.ds(k * 128, 128)
                pltpu.async_copy(row_sh.at[p_idx.at[sl]], pv.at[sl], gsem)
                pltpu.async_copy(row_sh.at[n_idx.at[sl]], nv.at[sl], gsem)
                return c2

            lax.fori_loop(0, BPT // 128, fire_i, 0)
            pltpu.make_async_copy(dummy, pv, gsem).wait()
            pltpu.make_async_copy(dummy, nv, gsem).wait()

            def accw(k, c3):
                sl = pl.ds(k * LANES, LANES)
                acc[sl] = acc[sl] + uv[sl] * (pv[sl] - nv[sl])
                return c3

            lax.fori_loop(0, BPT // LANES, accw, 0)
            plsc.subcore_barrier()
        return carry

    lax.fori_loop(0, NGRP, grp_body, 0)
    pltpu.sync_copy(acc, out_hbm.at[pl.ds(core * BATCH + tid * BPT, BPT)])


_sc_call = pl.kernel(
    _sc_body,
    mesh=plsc.VectorSubcoreMesh(core_axis_name="c", subcore_axis_name="s"),
    out_type=jax.ShapeDtypeStruct((NC * BATCH,), jnp.float32),
    scratch_types=[
        pltpu.VMEM_SHARED((NROWS,), jnp.float32),
        pltpu.VMEM((BPT,), jnp.int32),
        pltpu.VMEM((BPT,), jnp.int32),
        pltpu.VMEM((BPT,), jnp.int32),
        pltpu.VMEM((BPT,), jnp.float32),
        pltpu.VMEM((BPT,), jnp.float32),
        pltpu.VMEM((BPT,), jnp.float32),
        pltpu.VMEM((BPT,), jnp.float32),
        pltpu.VMEM((EMB * NTAIL,), jnp.float32),
        pltpu.VMEM((EMB * NTAIL,), jnp.float32),
        pltpu.SemaphoreType.DMA,
    ],
)


def _loss_body(x_ref, out_ref):
    x = x_ref[...]
    s = x[:128, :] + x[128:, :]
    l = jnp.log(jax.nn.sigmoid(s) + 1e-10)
    out_ref[0, 0] = -jnp.sum(l) * (1.0 / BATCH)


_loss_call = pl.pallas_call(
    _loss_body,
    out_shape=jax.ShapeDtypeStruct((1, 1), jnp.float32),
    out_specs=pl.BlockSpec(memory_space=pltpu.SMEM),
)


def kernel(u, pos, neg, u_emb_w, i_emb_w):
    u = u.astype(jnp.int32)
    pos = pos.astype(jnp.int32)
    neg = neg.astype(jnp.int32)
    uwt = u_emb_w.T.reshape(NC, NGRP, GRP, NROWS)
    iwt = i_emb_w.T.reshape(NC, NGRP, GRP, NROWS)
    aux_u = u_emb_w[TAIL2:, :].T.reshape(EMB * NTAIL)
    aux_i = i_emb_w[TAIL2:, :].T.reshape(EMB * NTAIL)
    part = _sc_call(u, pos, neg, uwt, iwt, aux_u, aux_i)
    loss = _loss_call(part.reshape(NC * BATCH // 128, 128))
    return loss[0, 0]


# speedup vs baseline: 2.6791x; 1.2189x over previous
import jax
import jax.numpy as jnp
from jax import lax
from jax.experimental import pallas as pl
from jax.experimental.pallas import tpu as pltpu
from jax.experimental.pallas import tpu_sc as plsc

BATCH = 16384
NROWS = 1000000
EMB = 64
LANES = 16
NC = 2
NS = 16
BPT = BATCH // NS
SEG = 62464
TAIL0 = SEG * NS
TAILW = 512
TAIL2 = TAIL0 + TAILW
NTAIL = NROWS - TAIL2
DPC = EMB // NC
GRP = 8
NGRP = DPC // GRP
NSPLIT = 8


def _sc_body(u_hbm, pos_hbm, neg_hbm, uwt_hbm, iwt_hbm, aux_u_hbm, aux_i_hbm,
             out_hbm,
             row_sh, u_idx, p_idx, n_idx, uv, pv, nv, acc,
             aux_u_v, aux_i_v, gsem, ssem):
    core = lax.axis_index("c")
    tid = lax.axis_index("s")

    bsl = pl.ds(tid * BPT, BPT)
    pltpu.sync_copy(u_hbm.at[bsl], u_idx)
    pltpu.sync_copy(pos_hbm.at[bsl], p_idx)
    pltpu.sync_copy(neg_hbm.at[bsl], n_idx)

    @pl.when(tid == NS - 1)
    def _stage_aux():
        pltpu.sync_copy(aux_u_hbm, aux_u_v)
        pltpu.sync_copy(aux_i_hbm, aux_i_v)

    def zero_body(k, carry):
        acc[pl.ds(k * LANES, LANES)] = jnp.zeros((LANES,), jnp.float32)
        return carry

    lax.fori_loop(0, BPT // LANES, zero_body, 0)

    start = pl.multiple_of(tid * SEG, 128)
    ssl = pl.ds(start, SEG)
    tsl = pl.ds(TAIL0, TAILW)
    dsl = pl.ds(TAIL2, NTAIL)
    dummy = aux_u_hbm.at[pl.ds(0, BPT)]

    def stage_row(table_hbm, aux_v, g, d8):
        cps = []
        for j in range(NSPLIT):
            sub = pl.ds(pl.multiple_of(start + j * (SEG // NSPLIT), 128),
                        SEG // NSPLIT)
            cps.append(pltpu.async_copy(
                table_hbm.at[core, g, d8, sub], row_sh.at[sub], ssem))

        @pl.when(tid == NS - 1)
        def _tail():
            pltpu.sync_copy(table_hbm.at[core, g, d8, tsl], row_sh.at[tsl])
            off = pl.multiple_of((core * DPC + g * GRP + d8) * NTAIL, NTAIL)
            pltpu.sync_copy(aux_v.at[pl.ds(off, NTAIL)], row_sh.at[dsl])

        for cp in cps:
            cp.wait()
        plsc.subcore_barrier()

    def grp_body(g, carry):
        for d8 in range(GRP):
            stage_row(uwt_hbm, aux_u_v, g, d8)

            def fire_u(k, c2):
                sl = pl.ds(k * 128, 128)
                pltpu.async_copy(row_sh.at[u_idx.at[sl]], uv.at[sl], gsem)
                return c2

            lax.fori_loop(0, BPT // 128, fire_u, 0)
            pltpu.make_async_copy(dummy, uv, gsem).wait()
            plsc.subcore_barrier()

            stage_row(iwt_hbm, aux_i_v, g, d8)

            def fire_i(k, c2):
                sl = pl.ds(k * 128, 128)
                pltpu.async_copy(row_sh.at[p_idx.at[sl]], pv.at[sl], gsem)
                pltpu.async_copy(row_sh.at[n_idx.at[sl]], nv.at[sl], gsem)
                return c2

            lax.fori_loop(0, BPT // 128, fire_i, 0)
            pltpu.make_async_copy(dummy, pv, gsem).wait()
            pltpu.make_async_copy(dummy, nv, gsem).wait()

            def accw(k, c3):
                sl = pl.ds(k * LANES, LANES)
                acc[sl] = acc[sl] + uv[sl] * (pv[sl] - nv[sl])
                return c3

            lax.fori_loop(0, BPT // LANES, accw, 0)
            plsc.subcore_barrier()
        return carry

    lax.fori_loop(0, NGRP, grp_body, 0)
    pltpu.sync_copy(acc, out_hbm.at[pl.ds(core * BATCH + tid * BPT, BPT)])


_sc_call = pl.kernel(
    _sc_body,
    mesh=plsc.VectorSubcoreMesh(core_axis_name="c", subcore_axis_name="s"),
    out_type=jax.ShapeDtypeStruct((NC * BATCH,), jnp.float32),
    scratch_types=[
        pltpu.VMEM_SHARED((NROWS,), jnp.float32),
        pltpu.VMEM((BPT,), jnp.int32),
        pltpu.VMEM((BPT,), jnp.int32),
        pltpu.VMEM((BPT,), jnp.int32),
        pltpu.VMEM((BPT,), jnp.float32),
        pltpu.VMEM((BPT,), jnp.float32),
        pltpu.VMEM((BPT,), jnp.float32),
        pltpu.VMEM((BPT,), jnp.float32),
        pltpu.VMEM((EMB * NTAIL,), jnp.float32),
        pltpu.VMEM((EMB * NTAIL,), jnp.float32),
        pltpu.SemaphoreType.DMA,
        pltpu.SemaphoreType.DMA,
    ],
)


def _loss_body(x_ref, out_ref):
    x = x_ref[...]
    s = x[:128, :] + x[128:, :]
    l = jnp.log(jax.nn.sigmoid(s) + 1e-10)
    out_ref[0, 0] = -jnp.sum(l) * (1.0 / BATCH)


_loss_call = pl.pallas_call(
    _loss_body,
    out_shape=jax.ShapeDtypeStruct((1, 1), jnp.float32),
    out_specs=pl.BlockSpec(memory_space=pltpu.SMEM),
)


def kernel(u, pos, neg, u_emb_w, i_emb_w):
    u = u.astype(jnp.int32)
    pos = pos.astype(jnp.int32)
    neg = neg.astype(jnp.int32)
    uwt = u_emb_w.T.reshape(NC, NGRP, GRP, NROWS)
    iwt = i_emb_w.T.reshape(NC, NGRP, GRP, NROWS)
    aux_u = u_emb_w[TAIL2:, :].T.reshape(EMB * NTAIL)
    aux_i = i_emb_w[TAIL2:, :].T.reshape(EMB * NTAIL)
    part = _sc_call(u, pos, neg, uwt, iwt, aux_u, aux_i)
    loss = _loss_call(part.reshape(NC * BATCH // 128, 128))
    return loss[0, 0]
